# Initial kernel scaffold; baseline (speedup 1.0000x reference)
import jax
import jax.numpy as jnp
from jax import lax
from jax.experimental import pallas as pl
from jax.experimental.pallas import tpu as pltpu
from jax.experimental.pallas import tpu_sc as plsc

N = 10000
D = 128
E = 320000
NC, NS, L = 2, 16, 16
NW = NC * NS
W = 136
R_PAD = 10240
CL = 128
NCH = 80
E_PAD = NW * NCH * CL
ROWS_PER_TILE = R_PAD // NS


def _sc_body(xaug, rows3, cols3, parts, idx_r, idx_c, buf0, buf1, acc,
             sem0, sem1):
    c = lax.axis_index("c")
    s = lax.axis_index("s")
    wid = c * NS + s

    pltpu.sync_copy(rows3.at[wid], idx_r)
    pltpu.sync_copy(cols3.at[wid], idx_c)

    base = s * ROWS_PER_TILE
    pltpu.sync_copy(xaug.at[pl.ds(N, 240)], acc.at[pl.ds(base, 240)])
    pltpu.sync_copy(xaug.at[pl.ds(N, 240)], acc.at[pl.ds(base + 240, 240)])
    pltpu.sync_copy(xaug.at[pl.ds(N, 160)], acc.at[pl.ds(base + 480, 160)])
    plsc.subcore_barrier()

    pltpu.async_copy(xaug.at[idx_c.at[0]], buf0, sem0)
    pltpu.async_copy(xaug.at[idx_c.at[1]], buf1, sem1)

    def body(i, carry):
        j0 = 2 * i
        pltpu.make_async_copy(xaug.at[idx_c.at[j0]], buf0, sem0).wait()
        pltpu.sync_copy(buf0, acc.at[idx_r.at[j0]], add=True)
        pltpu.async_copy(xaug.at[idx_c.at[j0 + 2]], buf0, sem0)
        pltpu.make_async_copy(xaug.at[idx_c.at[j0 + 1]], buf1, sem1).wait()
        pltpu.sync_copy(buf1, acc.at[idx_r.at[j0 + 1]], add=True)
        pltpu.async_copy(xaug.at[idx_c.at[j0 + 3]], buf1, sem1)
        return carry

    lax.fori_loop(0, NCH // 2 - 1, body, 0)

    pltpu.make_async_copy(xaug.at[idx_c.at[NCH - 2]], buf0, sem0).wait()
    pltpu.sync_copy(buf0, acc.at[idx_r.at[NCH - 2]], add=True)
    pltpu.make_async_copy(xaug.at[idx_c.at[NCH - 1]], buf1, sem1).wait()
    pltpu.sync_copy(buf1, acc.at[idx_r.at[NCH - 1]], add=True)

    plsc.subcore_barrier()
    pltpu.sync_copy(acc.at[pl.ds(base, ROWS_PER_TILE)],
                    parts.at[c, pl.ds(base, ROWS_PER_TILE)])


_sc_scatter = pl.kernel(
    _sc_body,
    out_type=jax.ShapeDtypeStruct((NC, R_PAD, W), jnp.float32),
    mesh=plsc.VectorSubcoreMesh(core_axis_name="c", subcore_axis_name="s",
                                num_cores=NC, num_subcores=NS),
    scratch_types=[
        pltpu.VMEM((NCH, CL), jnp.int32),
        pltpu.VMEM((NCH, CL), jnp.int32),
        pltpu.VMEM((CL, W), jnp.float32),
        pltpu.VMEM((CL, W), jnp.float32),
        pltpu.VMEM_SHARED((R_PAD, W), jnp.float32),
        pltpu.SemaphoreType.DMA,
        pltpu.SemaphoreType.DMA,
    ],
)


def _combine_body(rate_ref, xa_ref, p_ref, o_ref):
    x = xa_ref[:, :D]
    p0 = p_ref[0]
    p1 = p_ref[1]
    ssum = p0[:, :D] + p1[:, :D]
    deg = p0[:, D:D + 1] + p1[:, D:D + 1]
    deg = jnp.maximum(deg, 1.0)
    r = jnp.clip(rate_ref[0], 0.0, 0.2)
    o_ref[...] = jnp.maximum(x - r * (ssum / deg), 0.0)


_BR = 1280


def _combine(xaug, parts, rate):
    return pl.pallas_call(
        _combine_body,
        grid=(R_PAD // _BR,),
        in_specs=[
            pl.BlockSpec(memory_space=pltpu.SMEM),
            pl.BlockSpec((_BR, W), lambda i: (i, 0)),
            pl.BlockSpec((NC, _BR, W), lambda i: (0, i, 0)),
        ],
        out_specs=pl.BlockSpec((_BR, D), lambda i: (i, 0)),
        out_shape=jax.ShapeDtypeStruct((R_PAD, D), jnp.float32),
    )(rate, xaug, parts)


@jax.jit
def kernel(x, edge_index, rate):
    xaug = jnp.zeros((R_PAD, W), jnp.float32)
    xaug = xaug.at[:N, :D].set(x)
    xaug = xaug.at[:N, D].set(1.0)

    rows = jnp.concatenate(
        [edge_index[0], jnp.zeros((E_PAD - E,), jnp.int32)])
    cols = jnp.concatenate(
        [edge_index[1], jnp.full((E_PAD - E,), N, jnp.int32)])
    rows3 = rows.reshape(NW, NCH, CL)
    cols3 = cols.reshape(NW, NCH, CL)

    parts = _sc_scatter(xaug, rows3, cols3)
    out = _combine(xaug, parts, rate)
    return out[:N]


# speedup vs baseline: 3.3519x; 3.3519x over previous
import jax
import jax.numpy as jnp
from jax import lax
from jax.experimental import pallas as pl
from jax.experimental.pallas import tpu as pltpu
from jax.experimental.pallas import tpu_sc as plsc

N = 10000
D = 128
E = 320000
NC, NS, L = 2, 16, 16
NW = NC * NS
W = 136
R_PAD = 10240
CL = 64
NCH = 160
E_PAD = NW * NCH * CL
ROWS_PER_TILE = R_PAD // NS


def _sc_body(xaug, rows3, cols3, parts, idx_r, idx_c, buf0, buf1, acc,
             sem0, sem1):
    c = lax.axis_index("c")
    s = lax.axis_index("s")
    wid = c * NS + s

    pltpu.sync_copy(rows3.at[wid], idx_r)
    pltpu.sync_copy(cols3.at[wid], idx_c)

    base = s * ROWS_PER_TILE
    pltpu.sync_copy(xaug.at[pl.ds(N, 240)], acc.at[pl.ds(base, 240)])
    pltpu.sync_copy(xaug.at[pl.ds(N, 240)], acc.at[pl.ds(base + 240, 240)])
    pltpu.sync_copy(xaug.at[pl.ds(N, 160)], acc.at[pl.ds(base + 480, 160)])
    plsc.subcore_barrier()

    pltpu.async_copy(xaug.at[idx_c.at[0]], buf0, sem0)
    pltpu.async_copy(xaug.at[idx_c.at[1]], buf1, sem1)

    def body(i, carry):
        j0 = 2 * i
        pltpu.make_async_copy(xaug.at[idx_c.at[j0]], buf0, sem0).wait()
        pltpu.sync_copy(buf0, acc.at[idx_r.at[j0]], add=True)
        pltpu.async_copy(xaug.at[idx_c.at[j0 + 2]], buf0, sem0)
        pltpu.make_async_copy(xaug.at[idx_c.at[j0 + 1]], buf1, sem1).wait()
        pltpu.sync_copy(buf1, acc.at[idx_r.at[j0 + 1]], add=True)
        pltpu.async_copy(xaug.at[idx_c.at[j0 + 3]], buf1, sem1)
        return carry

    lax.fori_loop(0, NCH // 2 - 1, body, 0)

    pltpu.make_async_copy(xaug.at[idx_c.at[NCH - 2]], buf0, sem0).wait()
    pltpu.sync_copy(buf0, acc.at[idx_r.at[NCH - 2]], add=True)
    pltpu.make_async_copy(xaug.at[idx_c.at[NCH - 1]], buf1, sem1).wait()
    pltpu.sync_copy(buf1, acc.at[idx_r.at[NCH - 1]], add=True)

    plsc.subcore_barrier()
    pltpu.sync_copy(acc.at[pl.ds(base, ROWS_PER_TILE)],
                    parts.at[c, pl.ds(base, ROWS_PER_TILE)])


_sc_scatter = pl.kernel(
    _sc_body,
    out_type=jax.ShapeDtypeStruct((NC, R_PAD, W), jnp.float32),
    mesh=plsc.VectorSubcoreMesh(core_axis_name="c", subcore_axis_name="s",
                                num_cores=NC, num_subcores=NS),
    scratch_types=[
        pltpu.VMEM((NCH, CL), jnp.int32),
        pltpu.VMEM((NCH, CL), jnp.int32),
        pltpu.VMEM((CL, W), jnp.float32),
        pltpu.VMEM((CL, W), jnp.float32),
        pltpu.VMEM_SHARED((R_PAD, W), jnp.float32),
        pltpu.SemaphoreType.DMA,
        pltpu.SemaphoreType.DMA,
    ],
    compiler_params=pltpu.CompilerParams(use_tc_tiling_on_sc=False),
)


def _combine_body(rate_ref, xa_ref, p_ref, o_ref):
    x = xa_ref[:, :D]
    p0 = p_ref[0]
    p1 = p_ref[1]
    ssum = p0[:, :D] + p1[:, :D]
    deg = p0[:, D:D + 1] + p1[:, D:D + 1]
    deg = jnp.maximum(deg, 1.0)
    r = jnp.clip(rate_ref[0], 0.0, 0.2)
    o_ref[...] = jnp.maximum(x - r * (ssum / deg), 0.0)


_BR = 1280


def _combine(xaug, parts, rate):
    return pl.pallas_call(
        _combine_body,
        grid=(R_PAD // _BR,),
        in_specs=[
            pl.BlockSpec(memory_space=pltpu.SMEM),
            pl.BlockSpec((_BR, W), lambda i: (i, 0)),
            pl.BlockSpec((NC, _BR, W), lambda i: (0, i, 0)),
        ],
        out_specs=pl.BlockSpec((_BR, D), lambda i: (i, 0)),
        out_shape=jax.ShapeDtypeStruct((R_PAD, D), jnp.float32),
    )(rate, xaug, parts)


@jax.jit
def kernel(x, edge_index, rate):
    xaug = jnp.zeros((R_PAD, W), jnp.float32)
    xaug = xaug.at[:N, :D].set(x)
    xaug = xaug.at[:N, D].set(1.0)

    rows = jnp.concatenate(
        [edge_index[0], jnp.zeros((E_PAD - E,), jnp.int32)])
    cols = jnp.concatenate(
        [edge_index[1], jnp.full((E_PAD - E,), N, jnp.int32)])
    rows3 = rows.reshape(NW, NCH, CL)
    cols3 = cols.reshape(NW, NCH, CL)

    parts = _sc_scatter(xaug, rows3, cols3)
    out = _combine(xaug, parts, rate)
    return out[:N]


# speedup vs baseline: 7.5450x; 2.2510x over previous
import jax
import jax.numpy as jnp
from jax import lax
from jax.experimental import pallas as pl
from jax.experimental.pallas import tpu as pltpu
from jax.experimental.pallas import tpu_sc as plsc

N = 10000
D = 128
E = 320000
NC, NS, L = 2, 16, 16
NW = NC * NS
W = 136
R_PAD = 10240
CL = 64
NCH = 160
E_PAD = NW * NCH * CL
ROWS_PER_TILE = R_PAD // NS


def _sc_body(xaug, rows3, cols3, parts, idx_r, idx_c, buf0, buf1, acc,
             sem0, sem1):
    c = lax.axis_index("c")
    s = lax.axis_index("s")
    wid = c * NS + s

    pltpu.sync_copy(rows3.at[wid], idx_r)
    pltpu.sync_copy(cols3.at[wid], idx_c)

    base = s * ROWS_PER_TILE
    pltpu.sync_copy(xaug.at[pl.ds(N, 240)], acc.at[pl.ds(base, 240)])
    pltpu.sync_copy(xaug.at[pl.ds(N, 240)], acc.at[pl.ds(base + 240, 240)])
    pltpu.sync_copy(xaug.at[pl.ds(N, 160)], acc.at[pl.ds(base + 480, 160)])
    plsc.subcore_barrier()

    pltpu.async_copy(xaug.at[idx_c.at[0]], buf0, sem0)
    pltpu.async_copy(xaug.at[idx_c.at[1]], buf1, sem1)

    def body(i, carry):
        j0 = 2 * i
        pltpu.make_async_copy(xaug.at[idx_c.at[j0]], buf0, sem0).wait()
        pltpu.sync_copy(buf0, acc.at[idx_r.at[j0]], add=True)
        pltpu.async_copy(xaug.at[idx_c.at[j0 + 2]], buf0, sem0)
        pltpu.make_async_copy(xaug.at[idx_c.at[j0 + 1]], buf1, sem1).wait()
        pltpu.sync_copy(buf1, acc.at[idx_r.at[j0 + 1]], add=True)
        pltpu.async_copy(xaug.at[idx_c.at[j0 + 3]], buf1, sem1)
        return carry

    lax.fori_loop(0, NCH // 2 - 1, body, 0)

    pltpu.make_async_copy(xaug.at[idx_c.at[NCH - 2]], buf0, sem0).wait()
    pltpu.sync_copy(buf0, acc.at[idx_r.at[NCH - 2]], add=True)
    pltpu.make_async_copy(xaug.at[idx_c.at[NCH - 1]], buf1, sem1).wait()
    pltpu.sync_copy(buf1, acc.at[idx_r.at[NCH - 1]], add=True)

    plsc.subcore_barrier()
    pltpu.sync_copy(acc.at[pl.ds(base, ROWS_PER_TILE)],
                    parts.at[c, pl.ds(base, ROWS_PER_TILE)])


_sc_scatter = pl.kernel(
    _sc_body,
    out_type=jax.ShapeDtypeStruct((NC, R_PAD, W), jnp.float32),
    mesh=plsc.VectorSubcoreMesh(core_axis_name="c", subcore_axis_name="s",
                                num_cores=NC, num_subcores=NS),
    scratch_types=[
        pltpu.VMEM((NCH, CL), jnp.int32),
        pltpu.VMEM((NCH, CL), jnp.int32),
        pltpu.VMEM((CL, W), jnp.float32),
        pltpu.VMEM((CL, W), jnp.float32),
        pltpu.VMEM_SHARED((R_PAD, W), jnp.float32),
        pltpu.SemaphoreType.DMA,
        pltpu.SemaphoreType.DMA,
    ],
    compiler_params=pltpu.CompilerParams(use_tc_tiling_on_sc=False),
)


def _combine_body(rate_ref, xa_ref, p_ref, o_ref):
    x = xa_ref[:, :D]
    p0 = p_ref[0]
    p1 = p_ref[1]
    ssum = p0[:, :D] + p1[:, :D]
    deg = p0[:, D:D + 1] + p1[:, D:D + 1]
    deg = jnp.maximum(deg, 1.0)
    r = jnp.clip(rate_ref[0], 0.0, 0.2)
    o_ref[...] = jnp.maximum(x - r * (ssum / deg), 0.0)


_BR = 1280


def _combine(xaug, parts, rate):
    return pl.pallas_call(
        _combine_body,
        grid=(R_PAD // _BR,),
        in_specs=[
            pl.BlockSpec(memory_space=pltpu.SMEM),
            pl.BlockSpec((_BR, W), lambda i: (i, 0)),
            pl.BlockSpec((NC, _BR, W), lambda i: (0, i, 0)),
        ],
        out_specs=pl.BlockSpec((_BR, D), lambda i: (i, 0)),
        out_shape=jax.ShapeDtypeStruct((R_PAD, D), jnp.float32),
    )(rate, xaug, parts)


@jax.jit
def kernel(x, edge_index, rate):
    xaug = jnp.zeros((R_PAD, W), jnp.float32)
    xaug = xaug.at[:N, :D].set(x)
    xaug = xaug.at[:N, D].set(1.0)

    spread = N + (jax.lax.iota(jnp.int32, E_PAD - E) % (R_PAD - N))
    rows = jnp.concatenate([edge_index[0], spread])
    cols = jnp.concatenate([edge_index[1], spread])
    rows3 = rows.reshape(NW, NCH, CL)
    cols3 = cols.reshape(NW, NCH, CL)

    parts = _sc_scatter(xaug, rows3, cols3)
    out = _combine(xaug, parts, rate)
    return out[:N]


# speedup vs baseline: 10.0201x; 1.3280x over previous
import jax
import jax.numpy as jnp
from jax import lax
from jax.experimental import pallas as pl
from jax.experimental.pallas import tpu as pltpu
from jax.experimental.pallas import tpu_sc as plsc

N = 10000
D = 128
E = 320000
NC, NS, L = 2, 16, 16
NW = NC * NS
DW = 8
CL = 80
NCH = 125
RPT = N // NS
ZR = 64


def _sc_body(x_hbm, rows3, cols3, ones_hbm, zero_hbm, parts, degp,
             idx_r, idx_c, buf0, buf1, ones_v, acc, deg, sem0, sem1, semd):
    c = lax.axis_index("c")
    s = lax.axis_index("s")
    wid = c * NS + s

    pltpu.sync_copy(rows3.at[wid], idx_r)
    pltpu.sync_copy(cols3.at[wid], idx_c)
    pltpu.sync_copy(ones_hbm, ones_v)

    base = s * RPT
    for k in range(9):
        pltpu.sync_copy(zero_hbm, acc.at[pl.ds(base + k * ZR, ZR)])
        pltpu.sync_copy(zero_hbm.at[:, :DW], deg.at[pl.ds(base + k * ZR, ZR)])
    pltpu.sync_copy(zero_hbm.at[pl.ds(0, 49)], acc.at[pl.ds(base + 9 * ZR, 49)])
    pltpu.sync_copy(zero_hbm.at[pl.ds(0, 49), :DW],
                    deg.at[pl.ds(base + 9 * ZR, 49)])
    plsc.subcore_barrier()

    pltpu.async_copy(x_hbm.at[idx_c.at[0]], buf0, sem0)
    pltpu.async_copy(x_hbm.at[idx_c.at[1]], buf1, sem1)

    def step(j, buf, sem):
        pltpu.make_async_copy(x_hbm.at[idx_c.at[j]], buf, sem).wait()
        dsc = pltpu.async_copy(ones_v, deg.at[idx_r.at[j]], semd, add=True)
        pltpu.sync_copy(buf, acc.at[idx_r.at[j]], add=True)
        dsc.wait()

    def body(i, carry):
        j0 = 2 * i
        step(j0, buf0, sem0)
        pltpu.async_copy(x_hbm.at[idx_c.at[j0 + 2]], buf0, sem0)
        step(j0 + 1, buf1, sem1)
        pltpu.async_copy(x_hbm.at[idx_c.at[j0 + 3]], buf1, sem1)
        return carry

    lax.fori_loop(0, (NCH - 3) // 2, body, 0)

    step(NCH - 3, buf0, sem0)
    pltpu.async_copy(x_hbm.at[idx_c.at[NCH - 1]], buf0, sem0)
    step(NCH - 2, buf1, sem1)
    step(NCH - 1, buf0, sem0)

    plsc.subcore_barrier()
    pltpu.sync_copy(acc.at[pl.ds(base, RPT)], parts.at[c, pl.ds(base, RPT)])
    pltpu.sync_copy(deg.at[pl.ds(base, RPT)], degp.at[c, pl.ds(base, RPT)])


_sc_scatter = pl.kernel(
    _sc_body,
    out_type=[
        jax.ShapeDtypeStruct((NC, N, D), jnp.float32),
        jax.ShapeDtypeStruct((NC, N, DW), jnp.float32),
    ],
    mesh=plsc.VectorSubcoreMesh(core_axis_name="c", subcore_axis_name="s",
                                num_cores=NC, num_subcores=NS),
    scratch_types=[
        pltpu.VMEM((NCH, CL), jnp.int32),
        pltpu.VMEM((NCH, CL), jnp.int32),
        pltpu.VMEM((CL, D), jnp.float32),
        pltpu.VMEM((CL, D), jnp.float32),
        pltpu.VMEM((CL, DW), jnp.float32),
        pltpu.VMEM_SHARED((N, D), jnp.float32),
        pltpu.VMEM_SHARED((N, DW), jnp.float32),
        pltpu.SemaphoreType.DMA,
        pltpu.SemaphoreType.DMA,
        pltpu.SemaphoreType.DMA,
    ],
    compiler_params=pltpu.CompilerParams(use_tc_tiling_on_sc=False),
)


def _combine_body(rate_ref, x_ref, p_ref, d_ref, o_ref):
    x = x_ref[...]
    ssum = p_ref[0] + p_ref[1]
    deg = d_ref[0, :, :1] + d_ref[1, :, :1]
    deg = jnp.maximum(deg, 1.0)
    r = jnp.clip(rate_ref[0], 0.0, 0.2)
    o_ref[...] = jnp.maximum(x - r * (ssum / deg), 0.0)


_BR = 1000


def _combine(x, parts, degp, rate):
    return pl.pallas_call(
        _combine_body,
        grid=(N // _BR,),
        in_specs=[
            pl.BlockSpec(memory_space=pltpu.SMEM),
            pl.BlockSpec((_BR, D), lambda i: (i, 0)),
            pl.BlockSpec((NC, _BR, D), lambda i: (0, i, 0)),
            pl.BlockSpec((NC, _BR, DW), lambda i: (0, i, 0)),
        ],
        out_specs=pl.BlockSpec((_BR, D), lambda i: (i, 0)),
        out_shape=jax.ShapeDtypeStruct((N, D), jnp.float32),
    )(rate, x, parts, degp)


@jax.jit
def kernel(x, edge_index, rate):
    rows3 = edge_index[0].reshape(NW, NCH, CL)
    cols3 = edge_index[1].reshape(NW, NCH, CL)
    ones_hbm = jnp.tile(jnp.eye(1, DW, dtype=jnp.float32), (CL, 1))
    zero_hbm = jnp.zeros((ZR, D), jnp.float32)

    parts, degp = _sc_scatter(x, rows3, cols3, ones_hbm, zero_hbm)
    return _combine(x, parts, degp, rate)
